# baseline (speedup 1.0000x reference)
import functools

import jax
import jax.numpy as jnp
from jax import lax
from jax.experimental import pallas as pl
from jax.experimental.pallas import tpu as pltpu
from jax.experimental.pallas import tpu_sc as plsc

_BR = 248


def _flip_kernel(x_ref, sel_ref, o_ref):
    i = pl.program_id(0)
    rows = i * _BR + jax.lax.broadcasted_iota(jnp.int32, (_BR, 1), 0)
    hit = jnp.any(rows == sel_ref[...], axis=1, keepdims=True)
    sign = jnp.where(hit, -1.0, 1.0).astype(x_ref.dtype)
    o_ref[...] = x_ref[...] * sign


def _tc_flip(data, sel2d):
    n, l = data.shape
    return pl.pallas_call(
        _flip_kernel,
        grid=(pl.cdiv(n, _BR),),
        in_specs=[
            pl.BlockSpec((_BR, l), lambda i: (i, 0)),
            pl.BlockSpec(sel2d.shape, lambda i: (0, 0)),
        ],
        out_specs=pl.BlockSpec((_BR, l), lambda i: (i, 0)),
        out_shape=jax.ShapeDtypeStruct((n, l), data.dtype),
        compiler_params=pltpu.CompilerParams(
            dimension_semantics=("arbitrary",),
            vmem_limit_bytes=128 * 1024 * 1024,
        ),
    )(data, sel2d)


_PN = 1024
_L = 16384
_NW = 32
_RPW = _PN // _NW
_CR = 2
_NBUF = 2
_NCHUNK = _RPW // _CR

_mesh = plsc.VectorSubcoreMesh(core_axis_name="c", subcore_axis_name="s")


@functools.partial(
    pl.kernel,
    mesh=_mesh,
    out_type=jax.ShapeDtypeStruct((_PN, _L), jnp.float32),
    scratch_types=[
        pltpu.VMEM((_NBUF, _CR, _L), jnp.float32),
        pltpu.SemaphoreType.DMA((_NBUF,)),
        pltpu.SemaphoreType.DMA((_NBUF,)),
    ],
)
def _sc_copy(data_hbm, out_hbm, buf, in_sems, out_sems):
    wid = lax.axis_index("s") * 2 + lax.axis_index("c")
    base = wid * _RPW

    def start_in(c, b):
        pltpu.make_async_copy(
            data_hbm.at[pl.ds(base + c * _CR, _CR)], buf.at[b], in_sems.at[b]
        ).start()

    def wait_in(b):
        pltpu.make_async_copy(
            data_hbm.at[pl.ds(base, _CR)], buf.at[b], in_sems.at[b]
        ).wait()

    def start_out(c, b):
        pltpu.make_async_copy(
            buf.at[b], out_hbm.at[pl.ds(base + c * _CR, _CR)], out_sems.at[b]
        ).start()

    def wait_out(b):
        pltpu.make_async_copy(
            buf.at[b], out_hbm.at[pl.ds(base, _CR)], out_sems.at[b]
        ).wait()

    for b in range(_NBUF):
        start_in(b, b)

    def outer(g, _):
        for b in range(_NBUF):
            c = g * _NBUF + b
            wait_in(b)
            start_out(c, b)
            wait_out(b)

            @pl.when(c + _NBUF < _NCHUNK)
            def _():
                start_in(c + _NBUF, b)

        return 0

    lax.fori_loop(0, _NCHUNK // _NBUF, outer, 0)


def kernel(data, selection):
    sel2d = selection.astype(jnp.int32).reshape(1, -1)
    out = _tc_flip(data, sel2d)
    extra = _sc_copy(data)
    return out, extra


# speedup vs baseline: 1.3697x; 1.3697x over previous
import jax
import jax.numpy as jnp
from jax.experimental import pallas as pl
from jax.experimental.pallas import tpu as pltpu

_BR = 248


def _flip_kernel(x_ref, sel_ref, o_ref):
    i = pl.program_id(0)
    rows = i * _BR + jax.lax.broadcasted_iota(jnp.int32, (_BR, 1), 0)
    hit = jnp.any(rows == sel_ref[...], axis=1, keepdims=True)
    sign = jnp.where(hit, -1.0, 1.0).astype(x_ref.dtype)
    o_ref[...] = x_ref[...] * sign


def kernel(data, selection):
    n, l = data.shape
    sel2d = selection.astype(jnp.int32).reshape(1, -1)
    return pl.pallas_call(
        _flip_kernel,
        grid=(pl.cdiv(n, _BR),),
        in_specs=[
            pl.BlockSpec((_BR, l), lambda i: (i, 0)),
            pl.BlockSpec(sel2d.shape, lambda i: (0, 0)),
        ],
        out_specs=pl.BlockSpec((_BR, l), lambda i: (i, 0)),
        out_shape=jax.ShapeDtypeStruct((n, l), data.dtype),
        compiler_params=pltpu.CompilerParams(
            dimension_semantics=("arbitrary",),
            vmem_limit_bytes=128 * 1024 * 1024,
        ),
    )(data, sel2d)
